# baseline (speedup 1.0000x reference)
import jax
import jax.numpy as jnp
from jax.experimental import pallas as pl
from jax.experimental.pallas import tpu as pltpu

_BLK = 1024


def _fused_kernel(x_hbm, xt_ref, idt_ref, idb_ref, emb_ref, w_ref, b_ref,
                  out_ref, xbuf, sems):
    h = xt_ref.shape[1]
    i = pl.program_id(0)
    n = pl.num_programs(0)
    half = n * _BLK

    def copy(chunk, slot):
        return pltpu.make_async_copy(
            x_hbm.at[pl.ds(half + chunk * _BLK, _BLK), :],
            xbuf.at[slot], sems.at[slot])

    @pl.when(i == 0)
    def _():
        copy(0, 0).start()
        copy(1, 1).start()

    w1 = w_ref[:, :h]
    w2 = w_ref[:, h:]
    tab = jax.lax.dot_general(
        emb_ref[...], w2, (((1,), (1,)), ((), ())),
        preferred_element_type=jnp.float32)
    d = (tab[1] - tab[0])[None, :]
    base = tab[0][None, :] + b_ref[...]

    m_t = jax.lax.dot_general(
        xt_ref[...], w1, (((1,), (1,)), ((), ())),
        preferred_element_type=jnp.float32)
    out_ref[0] = m_t + idt_ref[...].astype(jnp.float32) * d + base

    slot = jax.lax.rem(i, 2)
    copy(i, slot).wait()
    m_b = jax.lax.dot_general(
        xbuf[slot], w1, (((1,), (1,)), ((), ())),
        preferred_element_type=jnp.float32)
    out_ref[1] = m_b + idb_ref[...].astype(jnp.float32) * d + base

    @pl.when(i + 2 < n)
    def _():
        copy(i + 2, slot).start()


def kernel(input, is_predicate_id, emb_table, W, b):
    B, S, H = input.shape
    NC, HD = W.shape
    R = B * S
    x = input.reshape(R, H)
    ids = is_predicate_id.reshape(R, 1).astype(jnp.int32)
    b2 = b.reshape(1, NC)
    n = R // (2 * _BLK)
    out = pl.pallas_call(
        _fused_kernel,
        grid=(n,),
        in_specs=[
            pl.BlockSpec(memory_space=pl.ANY),
            pl.BlockSpec((_BLK, H), lambda i: (i, 0)),
            pl.BlockSpec((_BLK, 1), lambda i: (i, 0)),
            pl.BlockSpec((_BLK, 1), lambda i, n=n: (i + n, 0)),
            pl.BlockSpec((2, HD - H), lambda i: (0, 0)),
            pl.BlockSpec((NC, HD), lambda i: (0, 0)),
            pl.BlockSpec((1, NC), lambda i: (0, 0)),
        ],
        out_specs=pl.BlockSpec((2, _BLK, NC), lambda i: (0, i, 0)),
        out_shape=jax.ShapeDtypeStruct((2, R // 2, NC), jnp.float32),
        scratch_shapes=[
            pltpu.VMEM((2, _BLK, H), jnp.float32),
            pltpu.SemaphoreType.DMA((2,)),
        ],
        compiler_params=pltpu.CompilerParams(
            dimension_semantics=("arbitrary",)),
    )(x, x, ids, ids, emb_table, W, b2)
    return out.reshape(B, S, NC)


# speedup vs baseline: 1.0185x; 1.0185x over previous
import jax
import jax.numpy as jnp
from jax.experimental import pallas as pl
from jax.experimental.pallas import tpu as pltpu

_BLK = 2048


def _fused_kernel(x_hbm, xt_ref, idt_ref, idb_ref, emb_ref, w_ref, b_ref,
                  out_ref, xbuf, sems):
    h = xt_ref.shape[1]
    i = pl.program_id(0)
    n = pl.num_programs(0)
    half = n * _BLK

    def copy(chunk, slot):
        return pltpu.make_async_copy(
            x_hbm.at[pl.ds(half + chunk * _BLK, _BLK), :],
            xbuf.at[slot], sems.at[slot])

    @pl.when(i == 0)
    def _():
        copy(0, 0).start()
        copy(1, 1).start()

    w1 = w_ref[:, :h]
    w2 = w_ref[:, h:]
    tab = jax.lax.dot_general(
        emb_ref[...], w2, (((1,), (1,)), ((), ())),
        preferred_element_type=jnp.float32)
    d = (tab[1] - tab[0])[None, :]
    base = tab[0][None, :] + b_ref[...]

    m_t = jax.lax.dot_general(
        xt_ref[...], w1, (((1,), (1,)), ((), ())),
        preferred_element_type=jnp.float32)
    out_ref[0] = m_t + idt_ref[...].astype(jnp.float32) * d + base

    slot = jax.lax.rem(i, 2)
    copy(i, slot).wait()
    m_b = jax.lax.dot_general(
        xbuf[slot], w1, (((1,), (1,)), ((), ())),
        preferred_element_type=jnp.float32)
    out_ref[1] = m_b + idb_ref[...].astype(jnp.float32) * d + base

    @pl.when(i + 2 < n)
    def _():
        copy(i + 2, slot).start()


def kernel(input, is_predicate_id, emb_table, W, b):
    B, S, H = input.shape
    NC, HD = W.shape
    R = B * S
    x = input.reshape(R, H)
    ids = is_predicate_id.reshape(R, 1).astype(jnp.int32)
    b2 = b.reshape(1, NC)
    n = R // (2 * _BLK)
    out = pl.pallas_call(
        _fused_kernel,
        grid=(n,),
        in_specs=[
            pl.BlockSpec(memory_space=pl.ANY),
            pl.BlockSpec((_BLK, H), lambda i: (i, 0)),
            pl.BlockSpec((_BLK, 1), lambda i: (i, 0)),
            pl.BlockSpec((_BLK, 1), lambda i, n=n: (i + n, 0)),
            pl.BlockSpec((2, HD - H), lambda i: (0, 0)),
            pl.BlockSpec((NC, HD), lambda i: (0, 0)),
            pl.BlockSpec((1, NC), lambda i: (0, 0)),
        ],
        out_specs=pl.BlockSpec((2, _BLK, NC), lambda i: (0, i, 0)),
        out_shape=jax.ShapeDtypeStruct((2, R // 2, NC), jnp.float32),
        scratch_shapes=[
            pltpu.VMEM((2, _BLK, H), jnp.float32),
            pltpu.SemaphoreType.DMA((2,)),
        ],
        compiler_params=pltpu.CompilerParams(
            dimension_semantics=("arbitrary",)),
    )(x, x, ids, ids, emb_table, W, b2)
    return out.reshape(B, S, NC)


# speedup vs baseline: 1.0624x; 1.0432x over previous
import jax
import jax.numpy as jnp
from jax.experimental import pallas as pl
from jax.experimental.pallas import tpu as pltpu

_BLK = 2048


def _fused_kernel(x_ref, ids_ref, emb_ref, w_ref, b_ref, out_ref):
    h = x_ref.shape[1]
    x = x_ref[...]
    w1 = w_ref[:, :h]
    w2 = w_ref[:, h:]
    tab = jax.lax.dot_general(
        emb_ref[...], w2, (((1,), (1,)), ((), ())),
        preferred_element_type=jnp.float32)
    m = jax.lax.dot_general(
        x, w1, (((1,), (1,)), ((), ())),
        preferred_element_type=jnp.float32)
    ids = ids_ref[...].astype(jnp.float32)
    contrib = tab[0][None, :] + ids * (tab[1] - tab[0])[None, :]
    out_ref[...] = m + contrib + b_ref[...]


def kernel(input, is_predicate_id, emb_table, W, b):
    B, S, H = input.shape
    NC, HD = W.shape
    R = B * S
    x = input.reshape(R, H)
    ids = is_predicate_id.reshape(R, 1).astype(jnp.int32)
    b2 = b.reshape(1, NC)
    grid = (R // _BLK,)
    out = pl.pallas_call(
        _fused_kernel,
        grid=grid,
        in_specs=[
            pl.BlockSpec((_BLK, H), lambda i: (i, 0)),
            pl.BlockSpec((_BLK, 1), lambda i: (i, 0)),
            pl.BlockSpec((2, HD - H), lambda i: (0, 0)),
            pl.BlockSpec((NC, HD), lambda i: (0, 0)),
            pl.BlockSpec((1, NC), lambda i: (0, 0)),
        ],
        out_specs=pl.BlockSpec((_BLK, NC), lambda i: (i, 0)),
        out_shape=jax.ShapeDtypeStruct((R, NC), jnp.float32),
        compiler_params=pltpu.CompilerParams(
            dimension_semantics=("arbitrary",)),
    )(x, ids, emb_table, W, b2)
    return out.reshape(B, S, NC)
